# Initial kernel scaffold; baseline (speedup 1.0000x reference)
import jax
import jax.numpy as jnp
from jax.experimental import pallas as pl


def kernel(x, edge_index, batch, params):
    raise NotImplementedError("write your pallas kernel here")


# speedup vs baseline: 4.1609x; 4.1609x over previous
import functools

import jax
import jax.numpy as jnp
from jax import lax
from jax.experimental import pallas as pl
from jax.experimental.pallas import tpu as pltpu
from jax.experimental.pallas import tpu_sc as plsc

_N = 10000
_D = 128
_E = 320000
_NC = 2
_NS = 16
_NW = _NC * _NS
_G = 79
_EPW = _G * 128
_EPAD = _NW * _EPW
_NPAD = 10112
_RPT = _NPAD // _NS
_NG = 64
_NCLS = 10


def _sc_segsum(x, src3, dst3):
    mesh = plsc.VectorSubcoreMesh(core_axis_name="c", subcore_axis_name="s")

    @functools.partial(
        pl.kernel,
        out_type=jax.ShapeDtypeStruct((_NC, _NPAD, _D), jnp.float32),
        mesh=mesh,
        scratch_types=[
            pltpu.VMEM((_G, 128), jnp.int32),
            pltpu.VMEM((_G, 128), jnp.int32),
            pltpu.VMEM((128, _D), jnp.float32),
            pltpu.VMEM_SHARED((_NPAD, _D), jnp.float32),
            pltpu.SemaphoreType.DMA,
        ],
    )
    def seg(x_hbm, src_hbm, dst_hbm, out_hbm, src_v, dst_v, rows_v, agg_sh, sem):
        c = lax.axis_index("c")
        s = lax.axis_index("s")
        wid = c * _NS + s

        pltpu.sync_copy(src_hbm.at[wid], src_v)
        pltpu.sync_copy(dst_hbm.at[wid], dst_v)

        zero = jnp.zeros((16,), jnp.float32)

        def zbody(i, carry):
            for jj in range(8):
                rows_v[i, pl.ds(jj * 16, 16)] = zero
            return carry

        lax.fori_loop(0, 128, zbody, 0)
        base = s * _RPT
        for k in range(4):
            pltpu.sync_copy(rows_v, agg_sh.at[pl.ds(base + k * 128, 128)])
        pltpu.sync_copy(rows_v.at[pl.ds(0, 120)],
                        agg_sh.at[pl.ds(base + 512, 120)])
        plsc.subcore_barrier()

        def ebody(j, carry):
            pltpu.async_copy(x_hbm.at[src_v.at[j]], rows_v, sem).wait()
            pltpu.sync_copy(rows_v, agg_sh.at[dst_v.at[j]], add=True)
            return carry

        lax.fori_loop(0, _G, ebody, 0)
        plsc.subcore_barrier()
        pltpu.sync_copy(agg_sh.at[pl.ds(base, _RPT)],
                        out_hbm.at[c, pl.ds(base, _RPT)])

    return seg(x, src3, dst3)


def _tc_layer_body(h_ref, agg_ref, w1_ref, b1_ref, g1_ref, bt1_ref,
                   w2_ref, b2_ref, g_ref, b_ref, out_ref, *, relu_out):
    z = h_ref[...] + agg_ref[0, :_N, :] + agg_ref[1, :_N, :]
    a = jnp.dot(z, w1_ref[...], preferred_element_type=jnp.float32) + b1_ref[...]
    m = jnp.mean(a, axis=0, keepdims=True)
    v = jnp.mean((a - m) * (a - m), axis=0, keepdims=True)
    a = (a - m) * lax.rsqrt(v + 1e-5) * g1_ref[...] + bt1_ref[...]
    a = jnp.maximum(a, 0.0)
    o = jnp.dot(a, w2_ref[...], preferred_element_type=jnp.float32) + b2_ref[...]
    m2 = jnp.mean(o, axis=0, keepdims=True)
    v2 = jnp.mean((o - m2) * (o - m2), axis=0, keepdims=True)
    o = (o - m2) * lax.rsqrt(v2 + 1e-5) * g_ref[...] + b_ref[...]
    if relu_out:
        o = jnp.maximum(o, 0.0)
    out_ref[...] = o


def _tc_layer(h, agg, conv, bn, relu_out):
    body = functools.partial(_tc_layer_body, relu_out=relu_out)
    return pl.pallas_call(
        body,
        out_shape=jax.ShapeDtypeStruct((_N, _D), jnp.float32),
    )(h, agg,
      conv['W1'], conv['b1'].reshape(1, -1), conv['g1'].reshape(1, -1),
      conv['bt1'].reshape(1, -1), conv['W2'], conv['b2'].reshape(1, -1),
      bn['g'].reshape(1, -1), bn['b'].reshape(1, -1))


def _tc_final_body(h_ref, agg_ref, w1_ref, b1_ref, g1_ref, bt1_ref,
                   w2_ref, b2_ref, g_ref, b_ref, batch_ref, wc_ref, bc_ref,
                   out_ref):
    z = h_ref[...] + agg_ref[0, :_N, :] + agg_ref[1, :_N, :]
    a = jnp.dot(z, w1_ref[...], preferred_element_type=jnp.float32) + b1_ref[...]
    m = jnp.mean(a, axis=0, keepdims=True)
    v = jnp.mean((a - m) * (a - m), axis=0, keepdims=True)
    a = (a - m) * lax.rsqrt(v + 1e-5) * g1_ref[...] + bt1_ref[...]
    a = jnp.maximum(a, 0.0)
    o = jnp.dot(a, w2_ref[...], preferred_element_type=jnp.float32) + b2_ref[...]
    m2 = jnp.mean(o, axis=0, keepdims=True)
    v2 = jnp.mean((o - m2) * (o - m2), axis=0, keepdims=True)
    o = (o - m2) * lax.rsqrt(v2 + 1e-5) * g_ref[...] + b_ref[...]
    gid = lax.broadcasted_iota(jnp.int32, (_N, _NG), 1)
    mask = (batch_ref[...] == gid).astype(jnp.float32)
    sums = lax.dot_general(mask, o, (((0,), (0,)), ((), ())),
                           preferred_element_type=jnp.float32)
    cnt = jnp.sum(mask, axis=0, keepdims=True)
    hg = sums / jnp.maximum(cnt, 1.0).reshape(_NG, 1)
    out_ref[...] = jnp.dot(hg, wc_ref[...],
                           preferred_element_type=jnp.float32) + bc_ref[...]


def _tc_final(h, agg, conv, bn, batch, cls):
    return pl.pallas_call(
        _tc_final_body,
        out_shape=jax.ShapeDtypeStruct((_NG, _NCLS), jnp.float32),
    )(h, agg,
      conv['W1'], conv['b1'].reshape(1, -1), conv['g1'].reshape(1, -1),
      conv['bt1'].reshape(1, -1), conv['W2'], conv['b2'].reshape(1, -1),
      bn['g'].reshape(1, -1), bn['b'].reshape(1, -1),
      batch.reshape(_N, 1), cls['W'], cls['b'].reshape(1, -1))


def kernel(x, edge_index, batch, params):
    pad = _EPAD - _E
    src3 = jnp.concatenate(
        [edge_index[0], jnp.zeros((pad,), jnp.int32)]).reshape(_NW, _G, 128)
    dst3 = jnp.concatenate(
        [edge_index[1], jnp.full((pad,), _N, jnp.int32)]).reshape(_NW, _G, 128)

    agg = _sc_segsum(x, src3, dst3)
    h = _tc_layer(x, agg, params['conv1'], params['bn1'], relu_out=True)
    agg = _sc_segsum(h, src3, dst3)
    h = _tc_layer(h, agg, params['convs'][0], params['bns'][0], relu_out=True)
    agg = _sc_segsum(h, src3, dst3)
    return _tc_final(h, agg, params['convs'][1], params['bns'][1],
                     batch, params['cls'])
